# baseline (speedup 1.0000x reference)
import functools

import jax
import jax.numpy as jnp
from jax import lax
from jax.experimental import pallas as pl
from jax.experimental.pallas import tpu as pltpu
from jax.experimental.pallas import tpu_sc as plsc

D = 64
L = 16
NC = 2
NS = 16
NW = NC * NS
GB = 128
K = 8
C = K * GB
UNROLL = 8
EPS = 1e-5


def _lane_perm(v, idx):
    dn = lax.GatherDimensionNumbers(
        offset_dims=(), collapsed_slice_dims=(0,), start_index_map=(0,)
    )
    return lax.gather(
        v, idx[:, None], dn, slice_sizes=(1,),
        mode=lax.GatherScatterMode.PROMISE_IN_BOUNDS,
    )


def _ln_body(idx_hbm, table_hbm, w_hbm, b_hbm, out_hbm, idx_v, rows_v, w_v, b_v, sem):
    wid = lax.axis_index("s") * NC + lax.axis_index("c")
    n_total = out_hbm.shape[0]
    per_w = n_total // NW
    n_chunks = per_w // C

    pltpu.sync_copy(w_hbm, w_v)
    pltpu.sync_copy(b_hbm, b_v)
    wregs = [w_v[pl.ds(L * k, L)] for k in range(D // L)]
    bregs = [b_v[pl.ds(L * k, L)] for k in range(D // L)]

    base128 = wid * (per_w // GB)
    lanes = lax.iota(jnp.int32, L)
    perm_idx = [lanes ^ d for d in (8, 4, 2, 1)]

    def chunk_body(c, carry):
        pltpu.sync_copy(idx_hbm.at[pl.ds(base128 + c * K, K)], idx_v)
        copies = [
            pltpu.async_copy(
                table_hbm.at[idx_v.at[j]], rows_v.at[pl.ds(j * GB, GB)], sem
            )
            for j in range(K)
        ]
        for cp in copies:
            cp.wait()

        def row_body(ri, rcarry):
            r0 = ri * UNROLL
            for u in range(UNROLL):
                r = r0 + u
                xs = [rows_v[r, pl.ds(L * k, L)] for k in range(D // L)]
                s = (xs[0] + xs[1]) + (xs[2] + xs[3])
                q = (xs[0] * xs[0] + xs[1] * xs[1]) + (xs[2] * xs[2] + xs[3] * xs[3])
                for pi in perm_idx:
                    s = s + _lane_perm(s, pi)
                    q = q + _lane_perm(q, pi)
                mean = s * (1.0 / D)
                var = q * (1.0 / D) - mean * mean
                rv = var + EPS
                iv = lax.bitcast_convert_type(rv, jnp.int32)
                y = lax.bitcast_convert_type(
                    jnp.int32(0x5F3759DF) - (iv >> 1), jnp.float32
                )
                y = y * (1.5 - 0.5 * rv * y * y)
                y = y * (1.5 - 0.5 * rv * y * y)
                for k in range(D // L):
                    rows_v[r, pl.ds(L * k, L)] = (xs[k] - mean) * y * wregs[k] + bregs[k]
            return rcarry

        lax.fori_loop(0, C // UNROLL, row_body, 0)
        pltpu.sync_copy(rows_v, out_hbm.at[pl.ds(wid * per_w + c * C, C)])
        return carry

    lax.fori_loop(0, n_chunks, chunk_body, 0)


def kernel(input_ids, table_value, ln_weight, ln_bias):
    n = input_ids.size
    idx2d = input_ids.reshape(n // GB, GB).astype(jnp.int32)
    mesh = plsc.VectorSubcoreMesh(core_axis_name="c", subcore_axis_name="s")
    run = functools.partial(
        pl.kernel,
        out_type=jax.ShapeDtypeStruct((n, D), jnp.float32),
        mesh=mesh,
        compiler_params=pltpu.CompilerParams(use_tc_tiling_on_sc=False),
        scratch_types=[
            pltpu.VMEM((K, GB), jnp.int32),
            pltpu.VMEM((C, D), jnp.float32),
            pltpu.VMEM((D,), jnp.float32),
            pltpu.VMEM((D,), jnp.float32),
            pltpu.SemaphoreType.DMA,
        ],
    )(_ln_body)
    out = run(idx2d, table_value, ln_weight, ln_bias)
    return out.reshape(input_ids.shape + (D,))


# speedup vs baseline: 1.0132x; 1.0132x over previous
import functools

import jax
import jax.numpy as jnp
from jax import lax
from jax.experimental import pallas as pl
from jax.experimental.pallas import tpu as pltpu
from jax.experimental.pallas import tpu_sc as plsc

D = 64
L = 16
NC = 2
NS = 16
NW = NC * NS
C = 1024
UNROLL = 4
EPS = 1e-5


def _lane_perm(v, idx):
    dn = lax.GatherDimensionNumbers(
        offset_dims=(), collapsed_slice_dims=(0,), start_index_map=(0,)
    )
    return lax.gather(
        v, idx[:, None], dn, slice_sizes=(1,),
        mode=lax.GatherScatterMode.PROMISE_IN_BOUNDS,
    )


def _ln_body(idx_hbm, table_hbm, w_hbm, b_hbm, out_hbm, idx_v, rows_v, w_v, b_v, sem):
    wid = lax.axis_index("s") * NC + lax.axis_index("c")
    n_total = out_hbm.shape[0]
    per_w = n_total // NW
    n_chunks = per_w // C

    pltpu.sync_copy(w_hbm, w_v)
    pltpu.sync_copy(b_hbm, b_v)
    wregs = [w_v[pl.ds(L * k, L)] for k in range(D // L)]
    bregs = [b_v[pl.ds(L * k, L)] for k in range(D // L)]

    base = wid * per_w
    lanes = lax.iota(jnp.int32, L)
    perm_idx = [lanes ^ d for d in (8, 4, 2, 1)]

    def chunk_body(c, carry):
        row0 = base + c * C
        pltpu.sync_copy(idx_hbm.at[pl.ds(row0, C)], idx_v)
        pltpu.async_copy(table_hbm.at[idx_v], rows_v, sem).wait()

        def row_body(ri, rcarry):
            r0 = ri * UNROLL
            for u in range(UNROLL):
                r = r0 + u
                xs = [rows_v[r, pl.ds(L * k, L)] for k in range(D // L)]
                s = (xs[0] + xs[1]) + (xs[2] + xs[3])
                q = (xs[0] * xs[0] + xs[1] * xs[1]) + (xs[2] * xs[2] + xs[3] * xs[3])
                for pi in perm_idx:
                    s = s + _lane_perm(s, pi)
                    q = q + _lane_perm(q, pi)
                mean = s * (1.0 / D)
                var = q * (1.0 / D) - mean * mean
                rv = var + EPS
                iv = lax.bitcast_convert_type(rv, jnp.int32)
                y = lax.bitcast_convert_type(
                    jnp.int32(0x5F3759DF) - (iv >> 1), jnp.float32
                )
                y = y * (1.5 - 0.5 * rv * y * y)
                y = y * (1.5 - 0.5 * rv * y * y)
                for k in range(D // L):
                    rows_v[r, pl.ds(L * k, L)] = (xs[k] - mean) * y * wregs[k] + bregs[k]
            return rcarry

        lax.fori_loop(0, C // UNROLL, row_body, 0)
        pltpu.sync_copy(rows_v, out_hbm.at[pl.ds(row0, C)])
        return carry

    lax.fori_loop(0, n_chunks, chunk_body, 0)


def kernel(input_ids, table_value, ln_weight, ln_bias):
    n = input_ids.size
    idx_flat = input_ids.reshape(n).astype(jnp.int32)
    mesh = plsc.VectorSubcoreMesh(core_axis_name="c", subcore_axis_name="s")
    run = functools.partial(
        pl.kernel,
        out_type=jax.ShapeDtypeStruct((n, D), jnp.float32),
        mesh=mesh,
        compiler_params=pltpu.CompilerParams(use_tc_tiling_on_sc=False),
        scratch_types=[
            pltpu.VMEM((C,), jnp.int32),
            pltpu.VMEM((C, D), jnp.float32),
            pltpu.VMEM((D,), jnp.float32),
            pltpu.VMEM((D,), jnp.float32),
            pltpu.SemaphoreType.DMA,
        ],
    )(_ln_body)
    out = run(idx_flat, table_value, ln_weight, ln_bias)
    return out.reshape(input_ids.shape + (D,))
